# baseline (speedup 1.0000x reference)
import functools

import jax
import jax.numpy as jnp
from jax import lax
from jax.experimental import pallas as pl
from jax.experimental.pallas import tpu as pltpu
from jax.experimental.pallas import tpu_sc as plsc

_B = 16384
_F = 128
_V = 100000
_ROWS_PER_BLOCK = 20480


def _tc_scores(user_row, item_table):
    nb = (_V + _ROWS_PER_BLOCK - 1) // _ROWS_PER_BLOCK
    vpad = nb * _ROWS_PER_BLOCK

    def body(u_ref, rows_ref, s_ref):
        s_ref[...] = lax.dot_general(
            u_ref[...], rows_ref[...], (((1,), (1,)), ((), ())),
            preferred_element_type=jnp.float32)

    out = pl.pallas_call(
        body,
        grid=(nb,),
        in_specs=[
            pl.BlockSpec((1, _F), lambda i: (0, 0)),
            pl.BlockSpec((_ROWS_PER_BLOCK, _F), lambda i: (i, 0)),
        ],
        out_specs=pl.BlockSpec((1, _ROWS_PER_BLOCK), lambda i: (0, i)),
        out_shape=jax.ShapeDtypeStruct((1, vpad), jnp.float32),
    )(user_row, item_table)
    return out.reshape(vpad)


def _sc_diff(pos_i, neg_j, scores):
    info = plsc.get_sparse_core_info()
    nc, ns = info.num_cores, info.num_subcores
    nw = nc * ns
    bpw = _B // nw
    mesh = plsc.VectorSubcoreMesh(core_axis_name="c", subcore_axis_name="s")

    @functools.partial(
        pl.kernel,
        mesh=mesh,
        compiler_params=pltpu.CompilerParams(needs_layout_passes=False),
        out_type=jax.ShapeDtypeStruct((_B,), jnp.float32),
        scratch_types=[
            pltpu.VMEM((bpw,), jnp.int32),
            pltpu.VMEM((bpw,), jnp.int32),
            pltpu.VMEM((bpw,), jnp.float32),
            pltpu.VMEM((bpw,), jnp.float32),
            pltpu.SemaphoreType.DMA,
            pltpu.SemaphoreType.DMA,
        ],
    )
    def body(pos_hbm, neg_hbm, s_hbm, out_hbm,
             idxp_v, idxn_v, sp_v, sn_v, sem0, sem1):
        wid = lax.axis_index("s") * nc + lax.axis_index("c")
        base = wid * bpw
        pltpu.sync_copy(pos_hbm.at[pl.ds(base, bpw)], idxp_v)
        pltpu.sync_copy(neg_hbm.at[pl.ds(base, bpw)], idxn_v)
        cp = pltpu.async_copy(s_hbm.at[idxp_v], sp_v, sem0)
        cn = pltpu.async_copy(s_hbm.at[idxn_v], sn_v, sem1)
        cp.wait()
        cn.wait()
        def diff(k, carry):
            sl = pl.ds(16 * k, 16)
            sp_v[sl] = sp_v[sl] - sn_v[sl]
            return carry

        lax.fori_loop(0, bpw // 16, diff, 0)
        pltpu.sync_copy(sp_v, out_hbm.at[pl.ds(base, bpw)])

    return body(pos_i, neg_j, scores)


def _tc_loss(x):
    def body(x_ref, o_ref):
        z = -x_ref[...]
        sp = jnp.maximum(z, 0.0) + jnp.log1p(jnp.exp(-jnp.abs(z)))
        o_ref[0, 0] = jnp.sum(sp)

    out = pl.pallas_call(
        body,
        out_shape=jax.ShapeDtypeStruct((1, 1), jnp.float32),
        out_specs=pl.BlockSpec(memory_space=pltpu.SMEM),
    )(x.reshape(_B // _F, _F))
    return out[0, 0]


def kernel(n_user, pos_i, neg_j, user_table, item_table):
    scores = _tc_scores(user_table, item_table)
    x = _sc_diff(pos_i, neg_j, scores)
    return _tc_loss(x)
